# baseline (speedup 1.0000x reference)
import jax
import jax.numpy as jnp
from jax import lax
from jax.experimental import pallas as pl
from jax.experimental.pallas import tpu as pltpu
from jax.experimental.pallas import tpu_sc as plsc

_N_EMBED = 16
_V = 1_000_000
_BATCH = 16384
_NC = 2
_NS = 16
_NW = _NC * _NS
_B_PER_W = _BATCH // _NW
_CHUNK = 128
_N_CHUNKS = _B_PER_W // _CHUNK

_G_COVER = 7808
_G_CAP = 7816
_TROW = _G_CAP * 1024
_STAGE_WORDS = 2 * _TROW
_V_COVER = _G_COVER * 128
_N_TAIL = _V - _V_COVER

_G_PER_W = _G_COVER * 2 // _NW
_SC_SIZES = [24] * 20 + [8]
_BUF_G = 24


def _stage_body(table_hbm, stage_hbm, buf0, buf1, isem0, isem1, osem):
    wid = lax.axis_index("s") * _NC + lax.axis_index("c")
    a = wid // 16
    g0 = (wid % 16) * _G_PER_W
    bufs = (buf0, buf1)
    isems = (isem0, isem1)
    offs = [0]
    for g in _SC_SIZES:
        offs.append(offs[-1] + g)
    n = len(_SC_SIZES)

    def issue_in(c):
        ng = _SC_SIZES[c]
        b = c % 2
        return [
            pltpu.async_copy(
                table_hbm.at[pl.ds(a * 8, 8),
                             pl.ds((g0 + offs[c] + g) * 128, 128)],
                bufs[b].at[g], isems[b])
            for g in range(ng)
        ]

    def issue_out(c):
        ng = _SC_SIZES[c]
        b = c % 2
        return pltpu.async_copy(
            bufs[b].at[pl.ds(0, ng)],
            stage_hbm.at[pl.ds(a * _G_CAP + g0 + offs[c], ng)], osem)

    cp_in = issue_in(0)
    prev_out = None
    for c in range(n):
        for cp in cp_in:
            cp.wait()
        if prev_out is not None:
            prev_out.wait()
        if c + 1 < n:
            cp_in = issue_in(c + 1)
        prev_out = issue_out(c)
    prev_out.wait()


def _gather_body(flat_hbm, idx_hbm, tail_hbm, out_hbm, idx_v, pidx_v, col_v,
                 tail_v, gsem, osem):
    wid = lax.axis_index("s") * _NC + lax.axis_index("c")
    base = wid * _B_PER_W
    pltpu.sync_copy(idx_hbm.at[pl.ds(base, _B_PER_W)], idx_v)
    pltpu.sync_copy(tail_hbm, tail_v)
    for k in range(_B_PER_W // 16):
        iv = idx_v[pl.ds(k * 16, 16)]
        safe = jnp.minimum(iv, _V_COVER - 1)
        p = ((safe >> 7) << 10) | (safe & 127)
        for e in range(_N_EMBED):
            const_e = (e // 8) * _TROW + (e % 8) * 128
            pidx_v[e, k // 8, pl.ds((k % 8) * 16, 16)] = p + const_e
    gathers = []
    for e in range(_N_EMBED):
        for j in range(_N_CHUNKS):
            gathers.append(
                pltpu.async_copy(
                    flat_hbm.at[pidx_v.at[e, j]],
                    col_v.at[e, pl.ds(j * _CHUNK, _CHUNK)],
                    gsem,
                ))
    for cp in gathers:
        cp.wait()
    for k in range(_B_PER_W // 16):
        iv = idx_v[pl.ds(k * 16, 16)]
        mask = iv >= _V_COVER
        loc = jnp.maximum(iv - _V_COVER, 0)
        for e in range(_N_EMBED):
            tv = plsc.load_gather(tail_v, [loc, jnp.full((16,), e, jnp.int32)])
            cur = col_v[e, pl.ds(k * 16, 16)]
            col_v[e, pl.ds(k * 16, 16)] = jnp.where(mask, tv, cur)
    outs = []
    for e in range(_N_EMBED):
        outs.append(
            pltpu.async_copy(
                col_v.at[e],
                out_hbm.at[pl.ds(e * _BATCH + base, _B_PER_W)],
                osem,
            ))
    for cp in outs:
        cp.wait()


@jax.jit
def _run(table_t, idx, tail):
    stage = pl.kernel(
        _stage_body,
        out_type=jax.ShapeDtypeStruct((2 * _G_CAP, 8, 128), jnp.float32),
        mesh=plsc.VectorSubcoreMesh(core_axis_name="c", subcore_axis_name="s"),
        compiler_params=pltpu.CompilerParams(needs_layout_passes=False),
        scratch_types=[
            pltpu.VMEM((_BUF_G, 8, 128), jnp.float32),
            pltpu.VMEM((_BUF_G, 8, 128), jnp.float32),
            pltpu.SemaphoreType.DMA,
            pltpu.SemaphoreType.DMA,
            pltpu.SemaphoreType.DMA,
        ],
    )
    staged = stage(table_t)
    flat = staged.reshape(_STAGE_WORDS)
    gather = pl.kernel(
        _gather_body,
        out_type=jax.ShapeDtypeStruct((_N_EMBED * _BATCH,), jnp.float32),
        mesh=plsc.VectorSubcoreMesh(core_axis_name="c", subcore_axis_name="s"),
        compiler_params=pltpu.CompilerParams(use_tc_tiling_on_sc=False,
                                             needs_layout_passes=False),
        scratch_types=[
            pltpu.VMEM((_B_PER_W,), jnp.int32),
            pltpu.VMEM((_N_EMBED, _N_CHUNKS, _CHUNK), jnp.int32),
            pltpu.VMEM((_N_EMBED, _B_PER_W), jnp.float32),
            pltpu.VMEM((_N_TAIL, _N_EMBED), jnp.float32),
            pltpu.SemaphoreType.DMA,
            pltpu.SemaphoreType.DMA,
        ],
    )
    return gather(flat, idx, tail)


def kernel(input_words, in_table):
    idx = input_words.astype(jnp.int32).reshape(_BATCH)
    tail = in_table[_V_COVER:, :]
    out_flat = _run(in_table.T, idx, tail)
    return out_flat.reshape(_N_EMBED, _BATCH).T
